# baseline (speedup 1.0000x reference)
import functools

import jax
import jax.numpy as jnp
from jax import lax
from jax.experimental import pallas as pl
from jax.experimental.pallas import tpu as pltpu
from jax.experimental.pallas import tpu_sc as plsc

VOCAB = 1000
VPAD = 1024
B = 1024
T = 200
N = B * T

NC = 2
NS = 16
NW = NC * NS
RW = N // NW
L = 16
NB = RW // L
CB = 8
NCB = VOCAB // CB


def _lse_body(table_ref, out_ref):
    t = table_ref[...]
    m = jnp.max(t, axis=1)
    s = jnp.sum(jnp.exp(t - m[:, None]), axis=1)
    out_ref[...] = m + jnp.log(s)


def _loss_body(part_ref, out_ref):
    s = jnp.sum(part_ref[...])
    out_ref[...] = jnp.full((1, 1), 1.0 / N, jnp.float32) * s


def _sc_body(table_t, x2, t2, lse, out_t, part,
             x_v, t_v, lse_v, trow, ob0, ob1, accd, macc, os0, os1):
    wid = lax.axis_index("s") * NC + lax.axis_index("c")
    i0 = wid * RW
    pltpu.sync_copy(x2.at[wid], x_v)
    pltpu.sync_copy(t2.at[wid], t_v)
    pltpu.sync_copy(lse, lse_v)
    accd[...] = jnp.zeros((L,), jnp.float32)
    macc[...] = jnp.zeros((L,), jnp.float32)

    def lse_batch(b, carry):
        xv = x_v[pl.ds(b * L, L)]
        accd[...] = accd[...] + plsc.load_gather(lse_v, [xv])
        return carry

    lax.fori_loop(0, NB, lse_batch, 0)

    def fill(cb, obuf):
        pltpu.sync_copy(table_t.at[pl.ds(cb * CB, CB)], trow)

        def batch(b, carry):
            xv = x_v[pl.ds(b * L, L)]
            tv = t_v[pl.ds(b * L, L)]
            for s in range(CB):
                c = cb * CB + s
                sv = jnp.full((L,), s, jnp.int32)
                gv = plsc.load_gather(trow, [sv, xv])
                obuf[s, pl.ds(b * L, L)] = gv
                macc[...] = macc[...] + jnp.where(tv == c, gv, 0.0)
            return carry

        lax.fori_loop(0, NB, batch, 0)

    def wb(cb, obuf, sem):
        return pltpu.async_copy(
            obuf, out_t.at[pl.ds(cb * CB, CB), pl.ds(i0, RW)], sem)

    def wb_wait(obuf, sem):
        pltpu.make_async_copy(
            obuf, out_t.at[pl.ds(0, CB), pl.ds(i0, RW)], sem).wait()

    def cpair(p, carry):
        cbA = 2 * p
        cbB = cbA + 1

        @pl.when(p > 0)
        def _():
            wb_wait(ob0, os0)

        fill(cbA, ob0)
        wb(cbA, ob0, os0)

        @pl.when(p > 0)
        def _():
            wb_wait(ob1, os1)

        fill(cbB, ob1)
        wb(cbB, ob1, os1)
        return carry

    lax.fori_loop(0, NCB // 2, cpair, 0)
    wb_wait(ob0, os0)
    fill(NCB - 1, ob0)
    wb(NCB - 1, ob0, os0)
    wb_wait(ob0, os0)
    wb_wait(ob1, os1)

    accd[...] = accd[...] - macc[...]
    pltpu.sync_copy(accd, part.at[wid])


def kernel(x, targets, table):
    x2 = x.reshape(NW, RW)
    t2 = targets.reshape(NW, RW)

    tpad = jnp.pad(table, ((0, VPAD - VOCAB), (0, VPAD - VOCAB)),
                   constant_values=-1e30)
    lse = pl.pallas_call(
        _lse_body,
        out_shape=jax.ShapeDtypeStruct((VPAD,), jnp.float32),
    )(tpad)

    table_t = tpad.T
    mesh = plsc.VectorSubcoreMesh(core_axis_name="c", subcore_axis_name="s")
    run = functools.partial(
        pl.kernel,
        out_type=[
            jax.ShapeDtypeStruct((VOCAB, N), jnp.float32),
            jax.ShapeDtypeStruct((NW, L), jnp.float32),
        ],
        mesh=mesh,
        compiler_params=pltpu.CompilerParams(
            needs_layout_passes=False, use_tc_tiling_on_sc=True),
        scratch_types=[
            pltpu.VMEM((RW,), jnp.int32),
            pltpu.VMEM((RW,), jnp.int32),
            pltpu.VMEM((VPAD,), jnp.float32),
            pltpu.VMEM((CB, VPAD), jnp.float32),
            pltpu.VMEM((CB, RW), jnp.float32),
            pltpu.VMEM((CB, RW), jnp.float32),
            pltpu.VMEM((L,), jnp.float32),
            pltpu.VMEM((L,), jnp.float32),
            pltpu.SemaphoreType.DMA,
            pltpu.SemaphoreType.DMA,
        ],
    )(_sc_body)
    out_t, part = run(table_t, x2, t2, lse)
    logits = out_t.T

    loss2 = pl.pallas_call(
        _loss_body,
        out_shape=jax.ShapeDtypeStruct((1, 1), jnp.float32),
    )(part)
    return logits, loss2[0, 0]


# speedup vs baseline: 1.9083x; 1.9083x over previous
import functools

import jax
import jax.numpy as jnp
from jax import lax
from jax.experimental import pallas as pl
from jax.experimental.pallas import tpu as pltpu
from jax.experimental.pallas import tpu_sc as plsc

VOCAB = 1000
VPAD = 1024
B = 1024
T = 200
N = B * T

NC = 2
NS = 16
NW = NC * NS
RW = N // NW
L = 16
NB = RW // L
CB = 8
NCB = VOCAB // CB


def _lse_body(table_ref, out_ref):
    t = table_ref[...]
    m = jnp.max(t, axis=1)
    s = jnp.sum(jnp.exp(t - m[:, None]), axis=1)
    out_ref[...] = m + jnp.log(s)


def _loss_body(part_ref, out_ref):
    s = jnp.sum(part_ref[...])
    out_ref[...] = jnp.full((1, 1), 1.0 / N, jnp.float32) * s


def _sc_body(table_f, x2, t2, lse, out_t, part,
             x_v, t_v, lse_v, trow, ob0, ob1, accd, os0, os1):
    wid = lax.axis_index("s") * NC + lax.axis_index("c")
    i0 = wid * RW
    pltpu.sync_copy(x2.at[wid], x_v)
    pltpu.sync_copy(t2.at[wid], t_v)
    pltpu.sync_copy(lse, lse_v)
    zero = jnp.zeros((L,), jnp.float32)

    def lse_batch(b, a):
        xv = x_v[pl.ds(b * L, L)]
        return a + plsc.load_gather(lse_v, [xv])

    lse_sum = lax.fori_loop(0, NB, lse_batch, zero)

    def fill(cb, obuf, ms):
        pltpu.sync_copy(table_f.at[pl.ds(cb * (CB * VPAD), CB * VPAD)], trow)

        def batch(b, m):
            acc = list(m)
            xv = x_v[pl.ds(b * L, L)]
            tv = t_v[pl.ds(b * L, L)]
            sv = tv - cb * CB
            for s in range(CB):
                gv = plsc.load_gather(trow, [xv + s * VPAD])
                obuf[s, pl.ds(b * L, L)] = gv
                acc[s % 4] = acc[s % 4] + jnp.where(sv == s, gv, 0.0)
            return tuple(acc)

        return lax.fori_loop(0, NB, batch, ms)

    def wb(cb, obuf, sem):
        return pltpu.async_copy(
            obuf, out_t.at[pl.ds(cb * CB, CB), pl.ds(i0, RW)], sem)

    def wb_wait(obuf, sem):
        pltpu.make_async_copy(
            obuf, out_t.at[pl.ds(0, CB), pl.ds(i0, RW)], sem).wait()

    def cpair(p, ms):
        cbA = 2 * p
        cbB = cbA + 1

        @pl.when(p > 0)
        def _():
            wb_wait(ob0, os0)

        ms = fill(cbA, ob0, ms)
        wb(cbA, ob0, os0)

        @pl.when(p > 0)
        def _():
            wb_wait(ob1, os1)

        ms = fill(cbB, ob1, ms)
        wb(cbB, ob1, os1)
        return ms

    ms = lax.fori_loop(0, NCB // 2, cpair, (zero, zero, zero, zero))
    wb_wait(ob0, os0)
    ms = fill(NCB - 1, ob0, ms)
    wb(NCB - 1, ob0, os0)
    wb_wait(ob0, os0)
    wb_wait(ob1, os1)

    accd[...] = lse_sum - (ms[0] + ms[1]) - (ms[2] + ms[3])
    pltpu.sync_copy(accd, part.at[wid])


def kernel(x, targets, table):
    x2 = x.reshape(NW, RW)
    t2 = targets.reshape(NW, RW)

    tpad = jnp.pad(table, ((0, VPAD - VOCAB), (0, VPAD - VOCAB)),
                   constant_values=-1e30)
    lse = pl.pallas_call(
        _lse_body,
        out_shape=jax.ShapeDtypeStruct((VPAD,), jnp.float32),
    )(tpad)

    table_f = tpad.T.reshape(-1)
    mesh = plsc.VectorSubcoreMesh(core_axis_name="c", subcore_axis_name="s")
    run = functools.partial(
        pl.kernel,
        out_type=[
            jax.ShapeDtypeStruct((VOCAB, N), jnp.float32),
            jax.ShapeDtypeStruct((NW, L), jnp.float32),
        ],
        mesh=mesh,
        compiler_params=pltpu.CompilerParams(
            needs_layout_passes=False, use_tc_tiling_on_sc=True),
        scratch_types=[
            pltpu.VMEM((RW,), jnp.int32),
            pltpu.VMEM((RW,), jnp.int32),
            pltpu.VMEM((VPAD,), jnp.float32),
            pltpu.VMEM((CB * VPAD,), jnp.float32),
            pltpu.VMEM((CB, RW), jnp.float32),
            pltpu.VMEM((CB, RW), jnp.float32),
            pltpu.VMEM((L,), jnp.float32),
            pltpu.SemaphoreType.DMA,
            pltpu.SemaphoreType.DMA,
        ],
    )(_sc_body)
    out_t, part = run(table_f, x2, t2, lse)
    logits = out_t.T

    loss2 = pl.pallas_call(
        _loss_body,
        out_shape=jax.ShapeDtypeStruct((1, 1), jnp.float32),
    )(part)
    return logits, loss2[0, 0]


# speedup vs baseline: 6.2833x; 3.2926x over previous
import functools

import jax
import jax.numpy as jnp
from jax import lax
from jax.experimental import pallas as pl
from jax.experimental.pallas import tpu as pltpu
from jax.experimental.pallas import tpu_sc as plsc

VOCAB = 1000
VPAD = 1024
B = 1024
T = 200
N = B * T

NC = 2
NS = 16
NW = NC * NS
RW = N // NW
L = 16
NB = RW // L
CB = 8
NCB = VOCAB // CB


def _lse_body(table_ref, out_ref):
    t = table_ref[...]
    m = jnp.max(t, axis=1)
    s = jnp.sum(jnp.exp(t - m[:, None]), axis=1)
    out_ref[...] = m + jnp.log(s)


def _loss_body(part_ref, out_ref):
    s = jnp.sum(part_ref[...])
    out_ref[...] = jnp.full((1, 1), 1.0 / N, jnp.float32) * s


def _sc_body(table_f, x2, t2, lse, out_t, part,
             x_v, t_v, lse_v, trow, ob0, ob1, accd, os0, os1):
    wid = lax.axis_index("s") * NC + lax.axis_index("c")
    i0 = wid * RW
    pltpu.sync_copy(x2.at[wid], x_v)
    pltpu.sync_copy(t2.at[wid], t_v)
    pltpu.sync_copy(lse, lse_v)
    zero = jnp.zeros((L,), jnp.float32)

    def lse_batch(b, a):
        xv = x_v[pl.ds(b * L, L)]
        return a + plsc.load_gather(lse_v, [xv])

    lse_sum = plsc.parallel_loop(0, NB, carry=zero, unroll=2)(lse_batch)

    def fill(cb, obuf, ms):
        pltpu.sync_copy(table_f.at[pl.ds(cb * (CB * VPAD), CB * VPAD)], trow)

        def batch(b, m):
            acc = list(m)
            xv = x_v[pl.ds(b * L, L)]
            tv = t_v[pl.ds(b * L, L)]
            sv = tv - cb * CB
            for s in range(CB):
                gv = plsc.load_gather(trow, [xv + s * VPAD])
                obuf[s, pl.ds(b * L, L)] = gv
                acc[s % 4] = acc[s % 4] + jnp.where(sv == s, gv, 0.0)
            return tuple(acc)

        return plsc.parallel_loop(0, NB, carry=ms, unroll=2)(batch)

    def wb(cb, obuf, sem):
        return pltpu.async_copy(
            obuf, out_t.at[pl.ds(cb * CB, CB), pl.ds(i0, RW)], sem)

    def wb_wait(obuf, sem):
        pltpu.make_async_copy(
            obuf, out_t.at[pl.ds(0, CB), pl.ds(i0, RW)], sem).wait()

    def cpair(p, ms):
        cbA = 2 * p
        cbB = cbA + 1

        @pl.when(p > 0)
        def _():
            wb_wait(ob0, os0)

        ms = fill(cbA, ob0, ms)
        wb(cbA, ob0, os0)

        @pl.when(p > 0)
        def _():
            wb_wait(ob1, os1)

        ms = fill(cbB, ob1, ms)
        wb(cbB, ob1, os1)
        return ms

    ms = lax.fori_loop(0, NCB // 2, cpair, (zero, zero, zero, zero))
    wb_wait(ob0, os0)
    ms = fill(NCB - 1, ob0, ms)
    wb(NCB - 1, ob0, os0)
    wb_wait(ob0, os0)
    wb_wait(ob1, os1)

    accd[...] = lse_sum - (ms[0] + ms[1]) - (ms[2] + ms[3])
    pltpu.sync_copy(accd, part.at[wid])


def kernel(x, targets, table):
    x2 = x.reshape(NW, RW)
    t2 = targets.reshape(NW, RW)

    tpad = jnp.pad(table, ((0, VPAD - VOCAB), (0, VPAD - VOCAB)),
                   constant_values=-1e30)
    lse = pl.pallas_call(
        _lse_body,
        out_shape=jax.ShapeDtypeStruct((VPAD,), jnp.float32),
    )(tpad)

    table_f = tpad.T.reshape(-1)
    mesh = plsc.VectorSubcoreMesh(core_axis_name="c", subcore_axis_name="s")
    run = functools.partial(
        pl.kernel,
        out_type=[
            jax.ShapeDtypeStruct((VOCAB, N), jnp.float32),
            jax.ShapeDtypeStruct((NW, L), jnp.float32),
        ],
        mesh=mesh,
        compiler_params=pltpu.CompilerParams(
            needs_layout_passes=False, use_tc_tiling_on_sc=True),
        scratch_types=[
            pltpu.VMEM((RW,), jnp.int32),
            pltpu.VMEM((RW,), jnp.int32),
            pltpu.VMEM((VPAD,), jnp.float32),
            pltpu.VMEM((CB * VPAD,), jnp.float32),
            pltpu.VMEM((CB, RW), jnp.float32),
            pltpu.VMEM((CB, RW), jnp.float32),
            pltpu.VMEM((L,), jnp.float32),
            pltpu.SemaphoreType.DMA,
            pltpu.SemaphoreType.DMA,
        ],
    )(_sc_body)
    out_t, part = run(table_f, x2, t2, lse)
    logits = out_t.T

    loss2 = pl.pallas_call(
        _loss_body,
        out_shape=jax.ShapeDtypeStruct((1, 1), jnp.float32),
    )(part)
    return logits, loss2[0, 0]
